# baseline (speedup 1.0000x reference)
import dataclasses
import functools

import jax
import jax.numpy as jnp
from jax import lax
from jax.experimental import pallas as pl
from jax.experimental.pallas import tpu as pltpu
from jax.experimental.pallas import tpu_sc as plsc

N_USERS = 5000
N_ITEMS = 5000
N = N_USERS + N_ITEMS
NP = 10240
D = 128
E = 320000
N_LAYERS = 3

NUM_CORES = 2
NUM_SUBCORES = 16
NUM_TILES = NUM_CORES * NUM_SUBCORES
EPT = E // NUM_TILES
CHUNK = 40
N_CHUNKS = EPT // CHUNK
NBUF = 4
EBUF = 8
MAIN = (N_CHUNKS // EBUF) * EBUF
ROWS_PER_TILE = NP // NUM_SUBCORES


def _sc_compiler_params():
  cp = pltpu.CompilerParams()
  if "needs_layout_passes" in pltpu.CompilerParams.__dataclass_fields__:
    cp = dataclasses.replace(cp, needs_layout_passes=False)
  return cp


def _sc_layer(x, idx_packed, vals, zeros_tile):
  mesh = plsc.VectorSubcoreMesh(
      core_axis_name="c", subcore_axis_name="s",
      num_cores=NUM_CORES, num_subcores=NUM_SUBCORES)

  @functools.partial(
      pl.kernel,
      out_type=jax.ShapeDtypeStruct((NUM_CORES, NP, D), jnp.float32),
      mesh=mesh,
      scratch_types=[
          pltpu.VMEM((EBUF, 2, CHUNK), jnp.int32),
          pltpu.VMEM((EBUF * CHUNK,), jnp.float32),
          pltpu.VMEM((NBUF, CHUNK, D), jnp.float32),
          pltpu.VMEM_SHARED((NP, D), jnp.float32),
          [pltpu.SemaphoreType.DMA] * NBUF,
          [pltpu.SemaphoreType.DMA] * NBUF,
          [pltpu.SemaphoreType.DMA] * EBUF,
          [pltpu.SemaphoreType.DMA] * EBUF,
      ],
      compiler_params=_sc_compiler_params(),
  )
  def layer_kernel(x_hbm, idx_hbm, val_hbm, z_hbm, out_hbm,
                   edges_v, vals_v, rows_v, acc_sh, gsems, ssems, esems,
                   vsems):
    cid = lax.axis_index("c")
    sid = lax.axis_index("s")
    wid = sid * NUM_CORES + cid

    pltpu.sync_copy(z_hbm, acc_sh.at[pl.ds(sid * ROWS_PER_TILE, ROWS_PER_TILE)])
    plsc.subcore_barrier()

    def edge_dma(be, c):
      return pltpu.make_async_copy(idx_hbm.at[wid].at[c], edges_v.at[be],
                                   esems[be])

    def val_dma(be, c):
      base = wid * EPT + c * CHUNK
      return pltpu.make_async_copy(val_hbm.at[pl.ds(base, CHUNK)],
                                   vals_v.at[pl.ds(be * CHUNK, CHUNK)],
                                   vsems[be])

    def gather(b, be, c):
      return pltpu.make_async_copy(x_hbm.at[edges_v.at[be].at[0]],
                                   rows_v.at[b], gsems[b])

    def scatter(b, be, c):
      return pltpu.make_async_copy(rows_v.at[b], acc_sh.at[edges_v.at[be].at[1]],
                                   ssems[b])

    def scale(b, be):
      iv0 = jnp.full((16,), be * CHUNK, jnp.int32)

      @pl.loop(0, CHUNK, init_carry=iv0, unroll=4)
      def _(r, iv):
        vv = plsc.load_gather(vals_v, [iv])
        for k in range(D // 16):
          sl = rows_v[b, r, pl.ds(k * 16, 16)]
          rows_v[b, r, pl.ds(k * 16, 16)] = sl * vv
        return iv + 1

    for c0 in range(4):
      edge_dma(c0, c0).start()
      val_dma(c0, c0).start()
    for c0 in range(2):
      edge_dma(c0, c0).wait()
      gather(c0, c0, c0).start()

    @pl.loop(0, MAIN, step=EBUF)
    def _(i):
      for be in range(EBUF):
        c = i + be
        b = be % NBUF
        gather(b, be, c).wait()
        val_dma(be, c).wait()
        scale(b, be)
        pltpu.async_copy(rows_v.at[b], acc_sh.at[edges_v.at[be].at[1]],
                         ssems[b], add=True)
        b2 = (b + 2) % NBUF
        be2 = (be + 2) % EBUF
        be4 = (be + 4) % EBUF
        be_prev2 = (be + 6) % EBUF
        if be < 2:
          @pl.when(i > 0)
          def _():
            scatter(b2, be_prev2, c - 2).wait()
        else:
          scatter(b2, be_prev2, c - 2).wait()
        if be < 6:
          edge_dma(be4, c + 4).start()
          val_dma(be4, c + 4).start()
        else:
          @pl.when(i < MAIN - EBUF)
          def _():
            edge_dma(be4, c + 4).start()
            val_dma(be4, c + 4).start()
        edge_dma(be2, c + 2).wait()
        gather(b2, be2, c + 2).start()

    for b, c in ((0, MAIN), (1, MAIN + 1)):
      gather(b, b, c).wait()
      val_dma(b, c).wait()
      scale(b, b)
      pltpu.async_copy(rows_v.at[b], acc_sh.at[edges_v.at[b].at[1]],
                       ssems[b], add=True)
    scatter(2, 2, MAIN - 2).wait()
    scatter(3, 3, MAIN - 1).wait()
    scatter(0, 0, MAIN).wait()
    scatter(1, 1, MAIN + 1).wait()

    plsc.subcore_barrier()
    rs = pl.ds(sid * ROWS_PER_TILE, ROWS_PER_TILE)
    pltpu.sync_copy(acc_sh.at[rs], out_hbm.at[cid].at[rs])

  return layer_kernel(x, idx_packed, vals, zeros_tile)


def _tc_combine(partials):
  def body(p_ref, o_ref):
    o_ref[...] = p_ref[0] + p_ref[1]

  return pl.pallas_call(
      body,
      out_shape=jax.ShapeDtypeStruct((NP, D), jnp.float32),
      grid=(8,),
      in_specs=[pl.BlockSpec((NUM_CORES, NP // 8, D), lambda i: (0, i, 0))],
      out_specs=pl.BlockSpec((NP // 8, D), lambda i: (i, 0)),
  )(partials)


def _tc_mean(x0, x1, x2, x3):
  def body(a_ref, b_ref, c_ref, d_ref, o_ref):
    o_ref[...] = (a_ref[...] + b_ref[...] + c_ref[...] + d_ref[...]) * 0.25

  spec = pl.BlockSpec((NP // 8, D), lambda i: (i, 0))
  return pl.pallas_call(
      body,
      out_shape=jax.ShapeDtypeStruct((NP, D), jnp.float32),
      grid=(8,),
      in_specs=[spec] * 4,
      out_specs=spec,
  )(x0, x1, x2, x3)


@jax.jit
def kernel(user_emb, item_emb, edge_index, edge_vals):
  src = edge_index[0].astype(jnp.int32).reshape(NUM_TILES, N_CHUNKS, CHUNK)
  dst = edge_index[1].astype(jnp.int32).reshape(NUM_TILES, N_CHUNKS, CHUNK)
  idx_packed = jnp.stack([src, dst], axis=2)
  vals = edge_vals.astype(jnp.float32)

  x0 = jnp.zeros((NP, D), jnp.float32)
  x0 = x0.at[:N_USERS].set(user_emb).at[N_USERS:N].set(item_emb)
  zeros_tile = jnp.zeros((ROWS_PER_TILE, D), jnp.float32)

  xs = [x0]
  x = x0
  for _ in range(N_LAYERS):
    partials = _sc_layer(x, idx_packed, vals, zeros_tile)
    x = _tc_combine(partials)
    xs.append(x)

  light_out = _tc_mean(*xs)
  return (light_out[:N_USERS], light_out[N_USERS:N])


# speedup vs baseline: 1.2319x; 1.2319x over previous
import dataclasses
import functools

import jax
import jax.numpy as jnp
from jax import lax
from jax.experimental import pallas as pl
from jax.experimental.pallas import tpu as pltpu
from jax.experimental.pallas import tpu_sc as plsc

N_USERS = 5000
N_ITEMS = 5000
N = N_USERS + N_ITEMS
NP = 10240
D = 128
E = 320000
N_LAYERS = 3

NUM_CORES = 2
NUM_SUBCORES = 16
NUM_TILES = NUM_CORES * NUM_SUBCORES
EPT = E // NUM_TILES
CHUNK = 80
N_CHUNKS = EPT // CHUNK
NBUF = 4
EBUF = 8
MAIN = (N_CHUNKS // EBUF) * EBUF
ROWS_PER_TILE = NP // NUM_SUBCORES


def _sc_compiler_params():
  cp = pltpu.CompilerParams()
  if "needs_layout_passes" in pltpu.CompilerParams.__dataclass_fields__:
    cp = dataclasses.replace(cp, needs_layout_passes=False)
  return cp


def _sc_layer(x, idx_packed, vals, zeros_tile):
  mesh = plsc.VectorSubcoreMesh(
      core_axis_name="c", subcore_axis_name="s",
      num_cores=NUM_CORES, num_subcores=NUM_SUBCORES)

  @functools.partial(
      pl.kernel,
      out_type=jax.ShapeDtypeStruct((NUM_CORES, NP, D), jnp.float32),
      mesh=mesh,
      scratch_types=[
          pltpu.VMEM((EBUF, 2, CHUNK), jnp.int32),
          pltpu.VMEM((EBUF * CHUNK,), jnp.float32),
          pltpu.VMEM((NBUF, CHUNK, D), jnp.float32),
          pltpu.VMEM_SHARED((NP, D), jnp.float32),
          [pltpu.SemaphoreType.DMA] * NBUF,
          [pltpu.SemaphoreType.DMA] * NBUF,
          [pltpu.SemaphoreType.DMA] * EBUF,
          [pltpu.SemaphoreType.DMA] * EBUF,
      ],
      compiler_params=_sc_compiler_params(),
  )
  def layer_kernel(x_hbm, idx_hbm, val_hbm, z_hbm, out_hbm,
                   edges_v, vals_v, rows_v, acc_sh, gsems, ssems, esems,
                   vsems):
    cid = lax.axis_index("c")
    sid = lax.axis_index("s")
    wid = sid * NUM_CORES + cid

    pltpu.sync_copy(z_hbm, acc_sh.at[pl.ds(sid * ROWS_PER_TILE, ROWS_PER_TILE)])
    plsc.subcore_barrier()

    def edge_dma(be, c):
      return pltpu.make_async_copy(idx_hbm.at[wid].at[c], edges_v.at[be],
                                   esems[be])

    def val_dma(be, c):
      base = wid * EPT + c * CHUNK
      return pltpu.make_async_copy(val_hbm.at[pl.ds(base, CHUNK)],
                                   vals_v.at[pl.ds(be * CHUNK, CHUNK)],
                                   vsems[be])

    def gather(b, be, c):
      return pltpu.make_async_copy(x_hbm.at[edges_v.at[be].at[0]],
                                   rows_v.at[b], gsems[b])

    def scatter(b, be, c):
      return pltpu.make_async_copy(rows_v.at[b], acc_sh.at[edges_v.at[be].at[1]],
                                   ssems[b])

    def scale(b, be):
      iv0 = jnp.full((16,), be * CHUNK, jnp.int32)

      @pl.loop(0, CHUNK, init_carry=iv0)
      def _(r, iv):
        vv = plsc.load_gather(vals_v, [iv])
        for k in range(D // 16):
          sl = rows_v[b, r, pl.ds(k * 16, 16)]
          rows_v[b, r, pl.ds(k * 16, 16)] = sl * vv
        return iv + 1

    LOOK = 2

    def visit(c, b, be, swait, issue8, issue4):
      gather(b, be, c).wait()
      val_dma(be, c).wait()
      scale(b, be)
      pltpu.async_copy(rows_v.at[b], acc_sh.at[edges_v.at[be].at[1]],
                       ssems[b], add=True)
      b4 = (b + LOOK) % NBUF
      be4 = (be + LOOK) % EBUF
      be8 = (be + 2 * LOOK) % EBUF
      be12 = (be + 3 * LOOK) % EBUF
      if swait is True:
        scatter(b4, be12, c - LOOK).wait()
      elif swait is not False:
        @pl.when(swait)
        def _():
          scatter(b4, be12, c - LOOK).wait()
      if issue8:
        edge_dma(be8, c + 2 * LOOK).start()
        val_dma(be8, c + 2 * LOOK).start()
      if issue4:
        edge_dma(be4, c + LOOK).wait()
        gather(b4, be4, c + LOOK).start()

    for c0 in range(2 * LOOK):
      edge_dma(c0, c0).start()
      val_dma(c0, c0).start()
    for c0 in range(LOOK):
      edge_dma(c0, c0).wait()
      gather(c0 % NBUF, c0, c0).start()

    @pl.loop(0, MAIN, step=EBUF)
    def _(i):
      for be in range(EBUF):
        c = i + be
        b = be % NBUF
        if be < LOOK:
          visit(c, b, be, swait=(i > 0), issue8=True, issue4=True)
        else:
          visit(c, b, be, swait=True, issue8=True, issue4=True)

    for c in range(MAIN, N_CHUNKS):
      visit(c, c % NBUF, c % EBUF, swait=True,
            issue8=(c + 2 * LOOK < N_CHUNKS), issue4=(c + LOOK < N_CHUNKS))
    for c in range(N_CHUNKS - LOOK, N_CHUNKS):
      scatter(c % NBUF, c % EBUF, c).wait()

    plsc.subcore_barrier()
    rs = pl.ds(sid * ROWS_PER_TILE, ROWS_PER_TILE)
    pltpu.sync_copy(acc_sh.at[rs], out_hbm.at[cid].at[rs])

  return layer_kernel(x, idx_packed, vals, zeros_tile)


def _tc_combine(partials):
  def body(p_ref, o_ref):
    o_ref[...] = p_ref[0] + p_ref[1]

  return pl.pallas_call(
      body,
      out_shape=jax.ShapeDtypeStruct((NP, D), jnp.float32),
      grid=(8,),
      in_specs=[pl.BlockSpec((NUM_CORES, NP // 8, D), lambda i: (0, i, 0))],
      out_specs=pl.BlockSpec((NP // 8, D), lambda i: (i, 0)),
  )(partials)


def _tc_mean(x0, x1, x2, x3):
  def body(a_ref, b_ref, c_ref, d_ref, o_ref):
    o_ref[...] = (a_ref[...] + b_ref[...] + c_ref[...] + d_ref[...]) * 0.25

  spec = pl.BlockSpec((NP // 8, D), lambda i: (i, 0))
  return pl.pallas_call(
      body,
      out_shape=jax.ShapeDtypeStruct((NP, D), jnp.float32),
      grid=(8,),
      in_specs=[spec] * 4,
      out_specs=spec,
  )(x0, x1, x2, x3)


@jax.jit
def kernel(user_emb, item_emb, edge_index, edge_vals):
  src = edge_index[0].astype(jnp.int32).reshape(NUM_TILES, N_CHUNKS, CHUNK)
  dst = edge_index[1].astype(jnp.int32).reshape(NUM_TILES, N_CHUNKS, CHUNK)
  idx_packed = jnp.stack([src, dst], axis=2)
  vals = edge_vals.astype(jnp.float32)

  x0 = jnp.zeros((NP, D), jnp.float32)
  x0 = x0.at[:N_USERS].set(user_emb).at[N_USERS:N].set(item_emb)
  zeros_tile = jnp.zeros((ROWS_PER_TILE, D), jnp.float32)

  xs = [x0]
  x = x0
  for _ in range(N_LAYERS):
    partials = _sc_layer(x, idx_packed, vals, zeros_tile)
    x = _tc_combine(partials)
    xs.append(x)

  light_out = _tc_mean(*xs)
  return (light_out[:N_USERS], light_out[N_USERS:N])


# speedup vs baseline: 1.2505x; 1.0151x over previous
import dataclasses
import functools

import jax
import jax.numpy as jnp
from jax import lax
from jax.experimental import pallas as pl
from jax.experimental.pallas import tpu as pltpu
from jax.experimental.pallas import tpu_sc as plsc

N_USERS = 5000
N_ITEMS = 5000
N = N_USERS + N_ITEMS
NP = 10240
D = 128
E = 320000
N_LAYERS = 3

NUM_CORES = 2
NUM_SUBCORES = 16
NUM_TILES = NUM_CORES * NUM_SUBCORES
EPT = E // NUM_TILES
CHUNK = 80
N_CHUNKS = EPT // CHUNK
NBUF = 4
EBUF = 8
MAIN = (N_CHUNKS // EBUF) * EBUF
ROWS_PER_TILE = NP // NUM_SUBCORES


def _sc_compiler_params():
  cp = pltpu.CompilerParams()
  if "needs_layout_passes" in pltpu.CompilerParams.__dataclass_fields__:
    cp = dataclasses.replace(cp, needs_layout_passes=False)
  return cp


def _sc_layer(x, idx_packed, vals, zeros_tile):
  mesh = plsc.VectorSubcoreMesh(
      core_axis_name="c", subcore_axis_name="s",
      num_cores=NUM_CORES, num_subcores=NUM_SUBCORES)

  @functools.partial(
      pl.kernel,
      out_type=jax.ShapeDtypeStruct((NUM_CORES, NP, D), jnp.float32),
      mesh=mesh,
      scratch_types=[
          pltpu.VMEM((EBUF, 2, CHUNK), jnp.int32),
          pltpu.VMEM((EBUF * CHUNK,), jnp.float32),
          pltpu.VMEM((NBUF, CHUNK, D), jnp.float32),
          pltpu.VMEM_SHARED((NP, D), jnp.float32),
          [pltpu.SemaphoreType.DMA] * NBUF,
          [pltpu.SemaphoreType.DMA] * NBUF,
          [pltpu.SemaphoreType.DMA] * EBUF,
          [pltpu.SemaphoreType.DMA] * EBUF,
      ],
      compiler_params=_sc_compiler_params(),
  )
  def layer_kernel(x_hbm, idx_hbm, val_hbm, z_hbm, out_hbm,
                   edges_v, vals_v, rows_v, acc_sh, gsems, ssems, esems,
                   vsems):
    cid = lax.axis_index("c")
    sid = lax.axis_index("s")
    wid = sid * NUM_CORES + cid

    pltpu.sync_copy(z_hbm, acc_sh.at[pl.ds(sid * ROWS_PER_TILE, ROWS_PER_TILE)])
    plsc.subcore_barrier()

    def edge_dma(be, c):
      return pltpu.make_async_copy(idx_hbm.at[wid].at[c], edges_v.at[be],
                                   esems[be])

    def val_dma(be, c):
      base = wid * EPT + c * CHUNK
      return pltpu.make_async_copy(val_hbm.at[pl.ds(base, CHUNK)],
                                   vals_v.at[pl.ds(be * CHUNK, CHUNK)],
                                   vsems[be])

    def gather(b, be, c):
      return pltpu.make_async_copy(x_hbm.at[edges_v.at[be].at[0]],
                                   rows_v.at[b], gsems[b])

    def scatter(b, be, c):
      return pltpu.make_async_copy(rows_v.at[b], acc_sh.at[edges_v.at[be].at[1]],
                                   ssems[b])

    def scale(b, be):
      iv0 = jnp.full((16,), be * CHUNK, jnp.int32)

      @pl.loop(0, CHUNK, init_carry=iv0)
      def _(r, iv):
        vv = plsc.load_gather(vals_v, [iv])
        for k in range(D // 16):
          sl = rows_v[b, r, pl.ds(k * 16, 16)]
          rows_v[b, r, pl.ds(k * 16, 16)] = sl * vv
        return iv + 1

    LOOK = 2

    def visit(c, b, be, swait, issue8, issue4):
      gather(b, be, c).wait()
      val_dma(be, c).wait()
      scale(b, be)
      pltpu.async_copy(rows_v.at[b], acc_sh.at[edges_v.at[be].at[1]],
                       ssems[b], add=True)
      b4 = (b + LOOK) % NBUF
      be4 = (be + LOOK) % EBUF
      be8 = (be + 2 * LOOK) % EBUF
      be12 = (be + 3 * LOOK) % EBUF
      if swait is True:
        scatter(b4, be12, c - LOOK).wait()
      elif swait is not False:
        @pl.when(swait)
        def _():
          scatter(b4, be12, c - LOOK).wait()
      if issue8:
        edge_dma(be8, c + 2 * LOOK).start()
        val_dma(be8, c + 2 * LOOK).start()
      if issue4:
        edge_dma(be4, c + LOOK).wait()
        gather(b4, be4, c + LOOK).start()

    for c0 in range(2 * LOOK):
      edge_dma(c0, c0).start()
      val_dma(c0, c0).start()
    for c0 in range(LOOK):
      edge_dma(c0, c0).wait()
      gather(c0 % NBUF, c0, c0).start()

    @pl.loop(0, MAIN, step=EBUF)
    def _(i):
      for be in range(EBUF):
        c = i + be
        b = be % NBUF
        if be < LOOK:
          visit(c, b, be, swait=(i > 0), issue8=True, issue4=True)
        else:
          visit(c, b, be, swait=True, issue8=True, issue4=True)

    for c in range(MAIN, N_CHUNKS):
      visit(c, c % NBUF, c % EBUF, swait=True,
            issue8=(c + 2 * LOOK < N_CHUNKS), issue4=(c + LOOK < N_CHUNKS))
    for c in range(N_CHUNKS - LOOK, N_CHUNKS):
      scatter(c % NBUF, c % EBUF, c).wait()

    plsc.subcore_barrier()
    rs = pl.ds(sid * ROWS_PER_TILE, ROWS_PER_TILE)
    pltpu.sync_copy(acc_sh.at[rs], out_hbm.at[cid].at[rs])

  return layer_kernel(x, idx_packed, vals, zeros_tile)


def _tc_combine(partials):
  def body(p_ref, o_ref):
    o_ref[...] = p_ref[0] + p_ref[1]

  return pl.pallas_call(
      body,
      out_shape=jax.ShapeDtypeStruct((NP, D), jnp.float32),
      grid=(8,),
      in_specs=[pl.BlockSpec((NUM_CORES, NP // 8, D), lambda i: (0, i, 0))],
      out_specs=pl.BlockSpec((NP // 8, D), lambda i: (i, 0)),
  )(partials)


def _tc_mean(x0, x1, x2, partials3):
  def body(a_ref, b_ref, c_ref, p_ref, o_ref):
    o_ref[...] = (a_ref[...] + b_ref[...] + c_ref[...]
                  + p_ref[0] + p_ref[1]) * 0.25

  spec = pl.BlockSpec((NP // 8, D), lambda i: (i, 0))
  pspec = pl.BlockSpec((NUM_CORES, NP // 8, D), lambda i: (0, i, 0))
  return pl.pallas_call(
      body,
      out_shape=jax.ShapeDtypeStruct((NP, D), jnp.float32),
      grid=(8,),
      in_specs=[spec, spec, spec, pspec],
      out_specs=spec,
  )(x0, x1, x2, partials3)


@jax.jit
def kernel(user_emb, item_emb, edge_index, edge_vals):
  src = edge_index[0].astype(jnp.int32).reshape(NUM_TILES, N_CHUNKS, CHUNK)
  dst = edge_index[1].astype(jnp.int32).reshape(NUM_TILES, N_CHUNKS, CHUNK)
  idx_packed = jnp.stack([src, dst], axis=2)
  vals = edge_vals.astype(jnp.float32)

  x0 = jnp.zeros((NP, D), jnp.float32)
  x0 = x0.at[:N_USERS].set(user_emb).at[N_USERS:N].set(item_emb)
  zeros_tile = jnp.zeros((ROWS_PER_TILE, D), jnp.float32)

  xs = [x0]
  x = x0
  for _ in range(N_LAYERS - 1):
    partials = _sc_layer(x, idx_packed, vals, zeros_tile)
    x = _tc_combine(partials)
    xs.append(x)
  partials3 = _sc_layer(x, idx_packed, vals, zeros_tile)

  light_out = _tc_mean(xs[0], xs[1], xs[2], partials3)
  return (light_out[:N_USERS], light_out[N_USERS:N])
